# baseline (speedup 1.0000x reference)
import functools

import jax
import jax.numpy as jnp
from jax import lax
from jax.experimental import pallas as pl
from jax.experimental.pallas import tpu as pltpu
from jax.experimental.pallas import tpu_sc as plsc

_B, _S = 4096, 50
_D = 64
_N = _B * _S
_NW = 32
_BPW = _N // _NW
_C = 800
_NCHUNK = _BPW // _C


def _make_gather():
    mesh = plsc.VectorSubcoreMesh(core_axis_name="c", subcore_axis_name="s")

    @functools.partial(
        pl.kernel,
        mesh=mesh,
        out_type=jax.ShapeDtypeStruct((_N, _D), jnp.float32),
        scratch_types=[
            pltpu.VMEM((_BPW,), jnp.int32),
            pltpu.VMEM((2, _C, _D), jnp.float32),
            pltpu.SemaphoreType.DMA,
            pltpu.SemaphoreType.DMA,
        ],
        compiler_params=pltpu.CompilerParams(use_tc_tiling_on_sc=False),
    )
    def gather_kernel(idx_hbm, table_hbm, out_hbm, idx_v, rows_v, gsem, ssem):
        wid = lax.axis_index("s") * 2 + lax.axis_index("c")
        base = wid * _BPW
        pltpu.sync_copy(idx_hbm.at[pl.ds(base, _BPW)], idx_v)

        def start_gather(k):
            return pltpu.async_copy(
                table_hbm.at[idx_v.at[pl.ds(k * _C, _C)]],
                rows_v.at[k % 2],
                gsem,
            )

        def start_scatter(k):
            return pltpu.async_copy(
                rows_v.at[k % 2],
                out_hbm.at[pl.ds(base + k * _C, _C)],
                ssem,
            )

        gathers = [None] * _NCHUNK
        scatters = [None] * _NCHUNK
        gathers[0] = start_gather(0)
        for k in range(_NCHUNK):
            if k + 1 < _NCHUNK:
                if k >= 1:
                    scatters[k - 1].wait()
                gathers[k + 1] = start_gather(k + 1)
            gathers[k].wait()
            scatters[k] = start_scatter(k)
        scatters[_NCHUNK - 2].wait()
        scatters[_NCHUNK - 1].wait()

    return gather_kernel


_gather = _make_gather()


@jax.jit
def kernel(token_ids, weight):
    flat_ids = token_ids.reshape(_N).astype(jnp.int32)
    out = _gather(flat_ids, weight)
    return out.reshape(_B, _S, _D)


# speedup vs baseline: 1.0002x; 1.0002x over previous
import functools

import jax
import jax.numpy as jnp
from jax import lax
from jax.experimental import pallas as pl
from jax.experimental.pallas import tpu as pltpu
from jax.experimental.pallas import tpu_sc as plsc

_B, _S = 4096, 50
_D = 64
_N = _B * _S
_NW = 32
_BPW = _N // _NW
_C = 400
_NBUF = 4
_NCHUNK = _BPW // _C


def _make_gather():
    mesh = plsc.VectorSubcoreMesh(core_axis_name="c", subcore_axis_name="s")

    @functools.partial(
        pl.kernel,
        mesh=mesh,
        out_type=jax.ShapeDtypeStruct((_N, _D), jnp.float32),
        scratch_types=[
            pltpu.VMEM((_BPW,), jnp.int32),
            pltpu.VMEM((_NBUF, _C, _D), jnp.float32),
            pltpu.SemaphoreType.DMA,
            pltpu.SemaphoreType.DMA,
        ],
        compiler_params=pltpu.CompilerParams(use_tc_tiling_on_sc=False),
    )
    def gather_kernel(idx_hbm, table_hbm, out_hbm, idx_v, rows_v, gsem, ssem):
        wid = lax.axis_index("s") * 2 + lax.axis_index("c")
        base = wid * _BPW
        pltpu.sync_copy(idx_hbm.at[pl.ds(base, _BPW)], idx_v)

        def start_gather(k):
            return pltpu.async_copy(
                table_hbm.at[idx_v.at[pl.ds(k * _C, _C)]],
                rows_v.at[k % _NBUF],
                gsem,
            )

        def start_scatter(k):
            return pltpu.async_copy(
                rows_v.at[k % _NBUF],
                out_hbm.at[pl.ds(base + k * _C, _C)],
                ssem,
            )

        gathers = [None] * _NCHUNK
        scatters = [None] * _NCHUNK
        for j in range(_NBUF - 1):
            gathers[j] = start_gather(j)
        for k in range(_NCHUNK):
            nxt = k + _NBUF - 1
            if nxt < _NCHUNK:
                if nxt >= _NBUF:
                    scatters[nxt - _NBUF].wait()
                gathers[nxt] = start_gather(nxt)
            gathers[k].wait()
            scatters[k] = start_scatter(k)
        for k in range(_NCHUNK - _NBUF, _NCHUNK):
            scatters[k].wait()

    return gather_kernel


_gather = _make_gather()


@jax.jit
def kernel(token_ids, weight):
    flat_ids = token_ids.reshape(_N).astype(jnp.int32)
    out = _gather(flat_ids, weight)
    return out.reshape(_B, _S, _D)
